# baseline (speedup 1.0000x reference)
import functools

import jax
import jax.numpy as jnp
from jax.experimental import pallas as pl
from jax.experimental.pallas import tpu as pltpu

TOPK = 2
E = 16
BLOCK_S = 512
NSTREAM = 1
SUB_S = BLOCK_S // NSTREAM


def _route_block(logits):
    lt = jnp.transpose(logits)
    m1 = jnp.max(lt, axis=0, keepdims=True)
    i1 = jnp.argmax(lt, axis=0).reshape(1, -1)
    eidx = jax.lax.broadcasted_iota(jnp.int32, lt.shape, 0)
    masked = jnp.where(eidx == i1, -jnp.inf, lt)
    m2 = jnp.max(masked, axis=0, keepdims=True)
    i2 = jnp.argmax(masked, axis=0).reshape(1, -1)

    e2 = jnp.exp(m2 - m1)
    denom = 1.0 + e2
    w1 = 1.0 / denom
    w2 = e2 / denom

    p = jnp.exp(lt - m1)
    p = p / jnp.sum(p, axis=0, keepdims=True)
    psum = jnp.sum(p, axis=1, keepdims=True)
    return w1, w2, i1, i2, psum


def _router_kernel(*refs, nb, nsb, inv_total):
    x_refs = refs[:NSTREAM]
    wt_ref = refs[NSTREAM]
    w1_ref, w2_ref, s1_ref, s2_ref, aux_ref, acc_ref = refs[NSTREAM + 1:]
    bi = pl.program_id(0)
    si = pl.program_id(1)
    w = wt_ref[...]
    dn = (((1,), (1,)), ((), ()))
    ptot = None
    for k in range(NSTREAM):
        lk = jax.lax.dot_general(x_refs[k][0], w, dn,
                                 preferred_element_type=jnp.float32)
        w1k, w2k, s1k, s2k, pk = _route_block(lk)
        sl = pl.ds(k * SUB_S, SUB_S)
        w1_ref[0, 0:1, sl] = w1k
        w2_ref[0, 0:1, sl] = w2k
        s1_ref[0, 0:1, sl] = s1k
        s2_ref[0, 0:1, sl] = s2k
        ptot = pk if ptot is None else ptot + pk

    @pl.when((bi == 0) & (si == 0))
    def _():
        acc_ref[...] = jnp.zeros_like(acc_ref)

    acc_ref[...] += ptot

    @pl.when((bi == nb - 1) & (si == nsb - 1))
    def _():
        mean_pe = acc_ref[...] * inv_total
        aux_ref[0] = jnp.sum(E * mean_pe * mean_pe)


def _x_spec(k, h):
    return pl.BlockSpec((1, SUB_S, h),
                        lambda bi, si, _k=k: (bi, NSTREAM * si + _k, 0))


def kernel(hidden_states, gate_weight):
    b, s, h = hidden_states.shape
    n = b * s
    nsb = s // BLOCK_S

    body = functools.partial(_router_kernel, nb=b, nsb=nsb,
                             inv_total=1.0 / n)
    plane = pl.BlockSpec((1, 1, BLOCK_S), lambda bi, si: (bi, 0, si))
    plane_shape_f = jax.ShapeDtypeStruct((b, 1, s), jnp.float32)
    plane_shape_i = jax.ShapeDtypeStruct((b, 1, s), jnp.int32)
    w1, w2, s1, s2, aux = pl.pallas_call(
        body,
        grid=(b, nsb),
        in_specs=[_x_spec(k, h) for k in range(NSTREAM)] + [
            pl.BlockSpec((E, h), lambda bi, si: (0, 0)),
        ],
        out_specs=[
            plane, plane, plane, plane,
            pl.BlockSpec(memory_space=pltpu.SMEM),
        ],
        out_shape=[
            plane_shape_f, plane_shape_f, plane_shape_i, plane_shape_i,
            jax.ShapeDtypeStruct((1,), jnp.float32),
        ],
        scratch_shapes=[pltpu.VMEM((E, 1), jnp.float32)],
    )(*([hidden_states] * NSTREAM + [gate_weight]))

    rw = jnp.stack([w1.reshape(b, s), w2.reshape(b, s)], axis=-1)
    sel = jnp.stack([s1.reshape(b, s), s2.reshape(b, s)], axis=-1)
    return (rw, sel, aux[0])


# speedup vs baseline: 1.2067x; 1.2067x over previous
import functools

import jax
import jax.numpy as jnp
from jax.experimental import pallas as pl
from jax.experimental.pallas import tpu as pltpu

TOPK = 2
E = 16
BLOCK_S = 1024
NSTREAM = 1
SUB_S = BLOCK_S // NSTREAM


def _route_block(logits):
    lt = jnp.transpose(logits)
    m1 = jnp.max(lt, axis=0, keepdims=True)
    i1 = jnp.argmax(lt, axis=0).reshape(1, -1)
    eidx = jax.lax.broadcasted_iota(jnp.int32, lt.shape, 0)
    masked = jnp.where(eidx == i1, -jnp.inf, lt)
    m2 = jnp.max(masked, axis=0, keepdims=True)
    i2 = jnp.argmax(masked, axis=0).reshape(1, -1)

    e2 = jnp.exp(m2 - m1)
    denom = 1.0 + e2
    w1 = 1.0 / denom
    w2 = e2 / denom

    p = jnp.exp(lt - m1)
    p = p / jnp.sum(p, axis=0, keepdims=True)
    psum = jnp.sum(p, axis=1, keepdims=True)
    return w1, w2, i1, i2, psum


def _router_kernel(*refs, nb, nsb, inv_total):
    x_refs = refs[:NSTREAM]
    wt_ref = refs[NSTREAM]
    w1_ref, w2_ref, s1_ref, s2_ref, aux_ref, acc_ref = refs[NSTREAM + 1:]
    bi = pl.program_id(0)
    si = pl.program_id(1)
    w = wt_ref[...]
    dn = (((1,), (1,)), ((), ()))
    ptot = None
    for k in range(NSTREAM):
        lk = jax.lax.dot_general(x_refs[k][0], w, dn,
                                 preferred_element_type=jnp.float32)
        w1k, w2k, s1k, s2k, pk = _route_block(lk)
        sl = pl.ds(k * SUB_S, SUB_S)
        w1_ref[0, 0:1, sl] = w1k
        w2_ref[0, 0:1, sl] = w2k
        s1_ref[0, 0:1, sl] = s1k
        s2_ref[0, 0:1, sl] = s2k
        ptot = pk if ptot is None else ptot + pk

    @pl.when((bi == 0) & (si == 0))
    def _():
        acc_ref[...] = jnp.zeros_like(acc_ref)

    acc_ref[...] += ptot

    @pl.when((bi == nb - 1) & (si == nsb - 1))
    def _():
        mean_pe = acc_ref[...] * inv_total
        aux_ref[0] = jnp.sum(E * mean_pe * mean_pe)


def _x_spec(k, h):
    return pl.BlockSpec((1, SUB_S, h),
                        lambda bi, si, _k=k: (bi, NSTREAM * si + _k, 0))


def kernel(hidden_states, gate_weight):
    b, s, h = hidden_states.shape
    n = b * s
    nsb = s // BLOCK_S

    body = functools.partial(_router_kernel, nb=b, nsb=nsb,
                             inv_total=1.0 / n)
    plane = pl.BlockSpec((1, 1, BLOCK_S), lambda bi, si: (bi, 0, si))
    plane_shape_f = jax.ShapeDtypeStruct((b, 1, s), jnp.float32)
    plane_shape_i = jax.ShapeDtypeStruct((b, 1, s), jnp.int32)
    w1, w2, s1, s2, aux = pl.pallas_call(
        body,
        grid=(b, nsb),
        in_specs=[_x_spec(k, h) for k in range(NSTREAM)] + [
            pl.BlockSpec((E, h), lambda bi, si: (0, 0)),
        ],
        out_specs=[
            plane, plane, plane, plane,
            pl.BlockSpec(memory_space=pltpu.SMEM),
        ],
        out_shape=[
            plane_shape_f, plane_shape_f, plane_shape_i, plane_shape_i,
            jax.ShapeDtypeStruct((1,), jnp.float32),
        ],
        scratch_shapes=[pltpu.VMEM((E, 1), jnp.float32)],
    )(*([hidden_states] * NSTREAM + [gate_weight]))

    rw = jnp.stack([w1.reshape(b, s), w2.reshape(b, s)], axis=-1)
    sel = jnp.stack([s1.reshape(b, s), s2.reshape(b, s)], axis=-1)
    return (rw, sel, aux[0])
